# baseline (speedup 1.0000x reference)
import jax
import jax.numpy as jnp
from jax import lax
from jax.experimental import pallas as pl
from jax.experimental.pallas import tpu as pltpu
from jax.experimental.pallas import tpu_sc as plsc

_NUM_USERS = 1000000
_CODE_LEN = 8
_EMB_DIM = 64
_BATCH = 4096

_LANES = 16
_NW = 32
_UPW = _BATCH // _NW
_CPW = _UPW * _CODE_LEN
_GSZ = 128
_NG = 2 * _CPW // _GSZ
_UPG = _GSZ // _CODE_LEN


def _body(uid_hbm, u2r_hbm, u2s_hbm, emb_hbm, out_hbm,
          uid_v, uidx, cflat, buf0, buf1, outbuf, emb0_v,
          sem_c, sem0, sem1):
    wid = lax.axis_index("s") * 2 + lax.axis_index("c")
    base = wid * _UPW

    pltpu.sync_copy(uid_hbm.at[pl.ds(base, _UPW)], uid_v)

    @pl.loop(0, _CPW // _LANES)
    def _build_idx(i):
        k = lax.iota(jnp.int32, _LANES) + i * _LANES
        uvals = plsc.load_gather(uid_v, [k >> 3])
        uidx[pl.ds(i * _LANES, _LANES)] = uvals * _CODE_LEN + (k & 7)

    copies = []
    for toff, tab in ((0, u2r_hbm), (_CPW, u2s_hbm)):
        for j in range(_CPW // _GSZ):
            copies.append(pltpu.async_copy(
                tab.at[uidx.at[pl.ds(j * _GSZ, _GSZ)]],
                cflat.at[pl.ds(toff + j * _GSZ, _GSZ)], sem_c))
    pltpu.sync_copy(emb_hbm.at[0], emb0_v)
    for cp in copies:
        cp.wait()

    e0 = [emb0_v[pl.ds(c * _LANES, _LANES)] for c in range(_EMB_DIM // _LANES)]

    def start_gather(g, buf, sem):
        pltpu.async_copy(emb_hbm.at[cflat.at[pl.ds(g * _GSZ, _GSZ)]], buf, sem)

    def accumulate(g, buf):
        t = g >> 3
        g8 = g & 7

        @pl.loop(0, _UPG)
        def _users(u):
            row0 = u * _CODE_LEN
            acc = [buf[row0, pl.ds(c * _LANES, _LANES)]
                   for c in range(_EMB_DIM // _LANES)]
            for l in range(1, _CODE_LEN):
                for c in range(_EMB_DIM // _LANES):
                    acc[c] = acc[c] + buf[row0 + l, pl.ds(c * _LANES, _LANES)]
            cb = g * _GSZ + row0
            cv = cflat[pl.ds(cb, _LANES)]
            zmask = (cv == 0) & (lax.iota(jnp.int32, _LANES) < _CODE_LEN)
            zf = jnp.sum(jnp.where(zmask, 1.0, 0.0).astype(jnp.float32))
            urow = g8 * _UPG + u
            for c in range(_EMB_DIM // _LANES):
                outbuf[urow, t, pl.ds(c * _LANES, _LANES)] = acc[c] - zf * e0[c]

    start_gather(0, buf0, sem0)

    @pl.loop(0, _NG, step=2)
    def _groups(g0):
        for b, (buf, sem, nbuf, nsem) in enumerate(
                ((buf0, sem0, buf1, sem1), (buf1, sem1, buf0, sem0))):
            g = g0 + b

            @pl.when(g + 1 < _NG)
            def _():
                start_gather(g + 1, nbuf, nsem)

            pltpu.make_async_copy(
                emb_hbm.at[pl.ds(0, _GSZ), pl.ds(0, _EMB_DIM)], buf, sem).wait()
            accumulate(g, buf)

    pltpu.sync_copy(outbuf, out_hbm.at[wid])


def kernel(user_ids, user2rec_code, user2src_code, code_embedding):
    mesh = plsc.VectorSubcoreMesh(core_axis_name="c", subcore_axis_name="s")
    out = pl.kernel(
        _body,
        out_type=jax.ShapeDtypeStruct((_NW, _UPW, 2, _EMB_DIM), jnp.float32),
        mesh=mesh,
        compiler_params=pltpu.CompilerParams(
            needs_layout_passes=False, use_tc_tiling_on_sc=False),
        scratch_types=[
            pltpu.VMEM((_UPW,), jnp.int32),
            pltpu.VMEM((_CPW,), jnp.int32),
            pltpu.VMEM((2 * _CPW + _LANES,), jnp.int32),
            pltpu.VMEM((_GSZ, _EMB_DIM), jnp.float32),
            pltpu.VMEM((_GSZ, _EMB_DIM), jnp.float32),
            pltpu.VMEM((_UPW, 2, _EMB_DIM), jnp.float32),
            pltpu.VMEM((_EMB_DIM,), jnp.float32),
            pltpu.SemaphoreType.DMA,
            pltpu.SemaphoreType.DMA,
            pltpu.SemaphoreType.DMA,
        ],
    )(user_ids, user2rec_code.reshape(-1), user2src_code.reshape(-1),
      code_embedding)
    return out.reshape(_BATCH, 2 * _EMB_DIM)


# speedup vs baseline: 1.5331x; 1.5331x over previous
import jax
import jax.numpy as jnp
from jax import lax
from jax.experimental import pallas as pl
from jax.experimental.pallas import tpu as pltpu
from jax.experimental.pallas import tpu_sc as plsc

_NUM_USERS = 1000000
_CODE_LEN = 8
_EMB_DIM = 64
_BATCH = 4096

_LANES = 16
_NW = 32
_UPW = _BATCH // _NW
_CPW = _UPW * _CODE_LEN
_GSZ = 128
_NG = 2 * _CPW // _GSZ
_UPG = _GSZ // _CODE_LEN
_NCH = _EMB_DIM // _LANES
_NUM_CODES_PAIR = 50000


def _body(uid_hbm, u2r_hbm, u2s_hbm, emb_hbm, out_hbm,
          uid_v, ucodes_r, ucodes_s, cflat, cpair, buf0, buf1, outbuf, emb0_v,
          sem_c, sem0, sem1):
    wid = lax.axis_index("s") * 2 + lax.axis_index("c")
    base = wid * _UPW

    pltpu.sync_copy(uid_hbm.at[pl.ds(base, _UPW)], uid_v)
    for u in range(_UPW):
        if u % _LANES == 0:
            uv = uid_v[pl.ds(u, _LANES)]
        s = uv[u % _LANES]
        pltpu.async_copy(u2r_hbm.at[s], ucodes_r.at[u], sem_c)
        pltpu.async_copy(u2s_hbm.at[s], ucodes_s.at[u], sem_c)
    pltpu.sync_copy(emb_hbm.at[0], emb0_v)
    pltpu.make_async_copy(u2r_hbm.at[pl.ds(0, _UPW)], ucodes_r, sem_c).wait()
    pltpu.make_async_copy(u2s_hbm.at[pl.ds(0, _UPW)], ucodes_s, sem_c).wait()

    @pl.loop(0, _CPW // _LANES)
    def _flatten(i):
        k = lax.iota(jnp.int32, _LANES) + i * _LANES
        r = k >> 3
        c = k & 7
        vr = plsc.load_gather(ucodes_r, [r, c])
        cflat[pl.ds(i * _LANES, _LANES)] = vr
        cpair[pl.ds(i * _LANES, _LANES)] = vr >> 1
        vs = plsc.load_gather(ucodes_s, [r, c])
        cflat[pl.ds(_CPW + i * _LANES, _LANES)] = vs
        cpair[pl.ds(_CPW + i * _LANES, _LANES)] = vs >> 1

    e0 = [emb0_v[pl.ds(c * _LANES, _LANES)] for c in range(_NCH)]

    def start_gather(g, buf, sem):
        pltpu.async_copy(emb_hbm.at[cpair.at[pl.ds(g * _GSZ, _GSZ)]], buf, sem)

    def accumulate(g, buf):
        t = g >> 3
        g8 = g & 7

        @pl.loop(0, _UPG)
        def _users(u):
            row0 = u * _CODE_LEN
            cb = g * _GSZ + row0
            cv = cflat[pl.ds(cb, _LANES)]
            hv = (cv & 1) * _EMB_DIM
            acc = [buf[row0, pl.ds(hv[0] + c * _LANES, _LANES)]
                   for c in range(_NCH)]
            for l in range(1, _CODE_LEN):
                for c in range(_NCH):
                    acc[c] = acc[c] + buf[row0 + l,
                                          pl.ds(hv[l] + c * _LANES, _LANES)]
            zmask = (cv == 0) & (lax.iota(jnp.int32, _LANES) < _CODE_LEN)
            zf = jnp.sum(jnp.where(zmask, 1.0, 0.0).astype(jnp.float32))
            urow = g8 * _UPG + u
            cbase = t * _EMB_DIM
            for c in range(_NCH):
                outbuf[urow, pl.ds(cbase + c * _LANES, _LANES)] = (
                    acc[c] - zf * e0[c])

    start_gather(0, buf0, sem0)

    @pl.loop(0, _NG, step=2)
    def _groups(g0):
        for b, (buf, sem, nbuf, nsem) in enumerate(
                ((buf0, sem0, buf1, sem1), (buf1, sem1, buf0, sem0))):
            g = g0 + b

            @pl.when(g + 1 < _NG)
            def _():
                start_gather(g + 1, nbuf, nsem)

            pltpu.make_async_copy(
                emb_hbm.at[pl.ds(0, _GSZ)], buf, sem).wait()
            accumulate(g, buf)

    pltpu.sync_copy(outbuf, out_hbm.at[wid])


def kernel(user_ids, user2rec_code, user2src_code, code_embedding):
    mesh = plsc.VectorSubcoreMesh(core_axis_name="c", subcore_axis_name="s")
    out = pl.kernel(
        _body,
        out_type=jax.ShapeDtypeStruct((_NW, _UPW, 2 * _EMB_DIM), jnp.float32),
        mesh=mesh,
        compiler_params=pltpu.CompilerParams(needs_layout_passes=False),
        scratch_types=[
            pltpu.VMEM((_UPW,), jnp.int32),
            pltpu.VMEM((_UPW, _CODE_LEN), jnp.int32),
            pltpu.VMEM((_UPW, _CODE_LEN), jnp.int32),
            pltpu.VMEM((2 * _CPW + _LANES,), jnp.int32),
            pltpu.VMEM((2 * _CPW,), jnp.int32),
            pltpu.VMEM((_GSZ, 2 * _EMB_DIM), jnp.float32),
            pltpu.VMEM((_GSZ, 2 * _EMB_DIM), jnp.float32),
            pltpu.VMEM((_UPW, 2 * _EMB_DIM), jnp.float32),
            pltpu.VMEM((2 * _EMB_DIM,), jnp.float32),
            pltpu.SemaphoreType.DMA,
            pltpu.SemaphoreType.DMA,
            pltpu.SemaphoreType.DMA,
        ],
    )(user_ids, user2rec_code, user2src_code,
      code_embedding.reshape(_NUM_CODES_PAIR, 2 * _EMB_DIM))
    return out.reshape(_BATCH, 2 * _EMB_DIM)


# speedup vs baseline: 1.5940x; 1.0398x over previous
import jax
import jax.numpy as jnp
from jax import lax
from jax.experimental import pallas as pl
from jax.experimental.pallas import tpu as pltpu
from jax.experimental.pallas import tpu_sc as plsc

_NUM_USERS = 1000000
_CODE_LEN = 8
_EMB_DIM = 64
_BATCH = 4096

_LANES = 16
_NW = 32
_UPW = _BATCH // _NW
_CPW = _UPW * _CODE_LEN
_GSZ = 128
_NG = 2 * _CPW // _GSZ
_UPG = _GSZ // _CODE_LEN
_NCH = _EMB_DIM // _LANES


def _body(uid_hbm, u2r_hbm, u2s_hbm, emb_hbm, out_hbm,
          uid_v, ucodes_r, ucodes_s, cflat, buf0, buf1, outbuf, emb0_v,
          sem_c, sem0, sem1):
    wid = lax.axis_index("s") * 2 + lax.axis_index("c")
    base = wid * _UPW

    pltpu.sync_copy(uid_hbm.at[pl.ds(base, _UPW)], uid_v)
    for u in range(_UPW):
        if u % _LANES == 0:
            uv = uid_v[pl.ds(u, _LANES)]
        s = uv[u % _LANES]
        pltpu.async_copy(u2r_hbm.at[s], ucodes_r.at[u], sem_c)
        pltpu.async_copy(u2s_hbm.at[s], ucodes_s.at[u], sem_c)
    pltpu.sync_copy(emb_hbm.at[0], emb0_v)
    pltpu.make_async_copy(u2r_hbm.at[pl.ds(0, _UPW)], ucodes_r, sem_c).wait()
    pltpu.make_async_copy(u2s_hbm.at[pl.ds(0, _UPW)], ucodes_s, sem_c).wait()

    @pl.loop(0, _CPW // _LANES)
    def _flatten(i):
        k = lax.iota(jnp.int32, _LANES) + i * _LANES
        r = k >> 3
        c = k & 7
        cflat[pl.ds(i * _LANES, _LANES)] = plsc.load_gather(ucodes_r, [r, c])
        cflat[pl.ds(_CPW + i * _LANES, _LANES)] = (
            plsc.load_gather(ucodes_s, [r, c]))

    e0 = [emb0_v[pl.ds(c * _LANES, _LANES)] for c in range(_NCH)]

    def start_gather(g, buf, sem):
        @pl.loop(0, _GSZ // _LANES)
        def _enq(j):
            cv = cflat[pl.ds(g * _GSZ + j * _LANES, _LANES)]
            for t in range(_LANES):
                pltpu.async_copy(emb_hbm.at[cv[t]], buf.at[j * _LANES + t],
                                 sem)

    def accumulate(g, buf):
        t = g >> 3
        g8 = g & 7

        @pl.loop(0, _UPG)
        def _users(u):
            row0 = u * _CODE_LEN
            acc = [buf[row0, pl.ds(c * _LANES, _LANES)]
                   for c in range(_NCH)]
            for l in range(1, _CODE_LEN):
                for c in range(_NCH):
                    acc[c] = acc[c] + buf[row0 + l, pl.ds(c * _LANES, _LANES)]
            cb = g * _GSZ + row0
            cv = cflat[pl.ds(cb, _LANES)]
            zmask = (cv == 0) & (lax.iota(jnp.int32, _LANES) < _CODE_LEN)
            zf = jnp.sum(jnp.where(zmask, 1.0, 0.0).astype(jnp.float32))
            urow = g8 * _UPG + u
            cbase = t * _EMB_DIM
            for c in range(_NCH):
                outbuf[urow, pl.ds(cbase + c * _LANES, _LANES)] = (
                    acc[c] - zf * e0[c])

    start_gather(0, buf0, sem0)

    @pl.loop(0, _NG, step=2)
    def _groups(g0):
        for b, (buf, sem, nbuf, nsem) in enumerate(
                ((buf0, sem0, buf1, sem1), (buf1, sem1, buf0, sem0))):
            g = g0 + b

            @pl.when(g + 1 < _NG)
            def _():
                start_gather(g + 1, nbuf, nsem)

            pltpu.make_async_copy(
                emb_hbm.at[pl.ds(0, _GSZ)], buf, sem).wait()
            accumulate(g, buf)

    pltpu.sync_copy(outbuf, out_hbm.at[wid])


def kernel(user_ids, user2rec_code, user2src_code, code_embedding):
    mesh = plsc.VectorSubcoreMesh(core_axis_name="c", subcore_axis_name="s")
    out = pl.kernel(
        _body,
        out_type=jax.ShapeDtypeStruct((_NW, _UPW, 2 * _EMB_DIM), jnp.float32),
        mesh=mesh,
        compiler_params=pltpu.CompilerParams(needs_layout_passes=False),
        scratch_types=[
            pltpu.VMEM((_UPW,), jnp.int32),
            pltpu.VMEM((_UPW, _CODE_LEN), jnp.int32),
            pltpu.VMEM((_UPW, _CODE_LEN), jnp.int32),
            pltpu.VMEM((2 * _CPW + _LANES,), jnp.int32),
            pltpu.VMEM((_GSZ, _EMB_DIM), jnp.float32),
            pltpu.VMEM((_GSZ, _EMB_DIM), jnp.float32),
            pltpu.VMEM((_UPW, 2 * _EMB_DIM), jnp.float32),
            pltpu.VMEM((_EMB_DIM,), jnp.float32),
            pltpu.SemaphoreType.DMA,
            pltpu.SemaphoreType.DMA,
            pltpu.SemaphoreType.DMA,
        ],
    )(user_ids, user2rec_code, user2src_code, code_embedding)
    return out.reshape(_BATCH, 2 * _EMB_DIM)


# speedup vs baseline: 10.4502x; 6.5558x over previous
import jax
import jax.numpy as jnp
from jax import lax
from jax.experimental import pallas as pl
from jax.experimental.pallas import tpu as pltpu
from jax.experimental.pallas import tpu_sc as plsc

_NUM_USERS = 1000000
_CODE_LEN = 8
_EMB_DIM = 64
_BATCH = 4096

_LANES = 16
_NW = 32
_UPW = _BATCH // _NW
_CPW = _UPW * _CODE_LEN
_WUS = 8
_NWV = _UPW // _WUS
_WCD = _WUS * _CODE_LEN
_NCH = _EMB_DIM // _LANES


def _body(uid_hbm, u2r_hbm, u2s_hbm, emb_hbm, out_hbm,
          uid_v, ucol_v, blk_r0, blk_s0, blk_r1, blk_s1,
          cflat, buf_r0, buf_r1, buf_s0, buf_s1, wbuf, emb0_v,
          sem_u0, sem_u1, sem_r0, sem_r1, sem_s0, sem_s1):
    wid = lax.axis_index("s") * 2 + lax.axis_index("c")
    base = wid * _UPW

    pltpu.sync_copy(uid_hbm.at[pl.ds(base, _UPW)], uid_v.at[pl.ds(0, _UPW)])
    pltpu.sync_copy(emb_hbm.at[0], emb0_v)
    e0 = [emb0_v[pl.ds(c * _LANES, _LANES)] for c in range(_NCH)]

    def fetch_wave(w, blk_r, blk_s, sem):
        uv = uid_v[pl.ds(w * _WUS, _LANES)]
        ucol_v[pl.ds(w * _WUS, _LANES)] = uv & 127
        for i in range(_WUS):
            s = pl.multiple_of(uv[i] & -128, 128)
            pltpu.async_copy(u2r_hbm.at[:, pl.ds(s, 128)], blk_r.at[i], sem)
            pltpu.async_copy(u2s_hbm.at[:, pl.ds(s, 128)], blk_s.at[i], sem)

    def drain_wave(blk_r, blk_s, sem):
        @pl.loop(0, _WUS)
        def _drain(i):
            pltpu.make_async_copy(
                u2r_hbm.at[:, pl.ds(0, 128)], blk_r.at[i], sem).wait()
            pltpu.make_async_copy(
                u2r_hbm.at[:, pl.ds(0, 128)], blk_s.at[i], sem).wait()

    def extract_wave(w, blk_r, blk_s):
        @pl.loop(0, _WCD // _LANES)
        def _ext(i):
            k = lax.iota(jnp.int32, _LANES) + w * _WCD + i * _LANES
            u = k >> 3
            uu = u - w * _WUS
            l = k & 7
            col = plsc.load_gather(ucol_v, [u])
            cflat[pl.ds(w * _WCD + i * _LANES, _LANES)] = (
                plsc.load_gather(blk_r, [uu, l, col]))
            cflat[pl.ds(_CPW + w * _WCD + i * _LANES, _LANES)] = (
                plsc.load_gather(blk_s, [uu, l, col]))

    def enqueue_emb(cbase, buf, sem):
        @pl.loop(0, _WCD // _LANES)
        def _enq(j):
            cv = cflat[pl.ds(cbase + j * _LANES, _LANES)]
            for t in range(_LANES):
                pltpu.async_copy(emb_hbm.at[cv[t]], buf.at[j * _LANES + t],
                                 sem)

    def wait_emb(buf, sem):
        pltpu.make_async_copy(emb_hbm.at[pl.ds(0, _WCD)], buf, sem).wait()

    def accumulate(w, cbase, buf, colbase):
        @pl.loop(0, _WUS)
        def _users(u):
            row0 = u * _CODE_LEN
            acc = [buf[row0, pl.ds(c * _LANES, _LANES)]
                   for c in range(_NCH)]
            for l in range(1, _CODE_LEN):
                for c in range(_NCH):
                    acc[c] = acc[c] + buf[row0 + l, pl.ds(c * _LANES, _LANES)]
            cv = cflat[pl.ds(cbase + row0, _LANES)]
            zmask = (cv == 0) & (lax.iota(jnp.int32, _LANES) < _CODE_LEN)
            zf = jnp.sum(jnp.where(zmask, 1.0, 0.0).astype(jnp.float32))
            for c in range(_NCH):
                wbuf[u, pl.ds(colbase + c * _LANES, _LANES)] = (
                    acc[c] - zf * e0[c])

    fetch_wave(0, blk_r0, blk_s0, sem_u0)

    @pl.loop(0, _NWV, step=2)
    def _waves(w0):
        for b, (blk_r, blk_s, sem_u, nblk_r, nblk_s, nsem_u,
                buf_r, sem_r, buf_s, sem_s, obuf_r, osem_r, obuf_s, osem_s) \
                in enumerate((
                    (blk_r0, blk_s0, sem_u0, blk_r1, blk_s1, sem_u1,
                     buf_r0, sem_r0, buf_s0, sem_s0,
                     buf_r1, sem_r1, buf_s1, sem_s1),
                    (blk_r1, blk_s1, sem_u1, blk_r0, blk_s0, sem_u0,
                     buf_r1, sem_r1, buf_s1, sem_s1,
                     buf_r0, sem_r0, buf_s0, sem_s0))):
            w = w0 + b

            @pl.when(w + 1 < _NWV)
            def _():
                fetch_wave(w + 1, nblk_r, nblk_s, nsem_u)

            drain_wave(blk_r, blk_s, sem_u)
            extract_wave(w, blk_r, blk_s)
            enqueue_emb(w * _WCD, buf_r, sem_r)
            enqueue_emb(_CPW + w * _WCD, buf_s, sem_s)

            @pl.when(w > 0)
            def _():
                wait_emb(obuf_r, osem_r)
                accumulate(w - 1, (w - 1) * _WCD, obuf_r, 0)
                wait_emb(obuf_s, osem_s)
                accumulate(w - 1, _CPW + (w - 1) * _WCD, obuf_s, _EMB_DIM)
                pltpu.sync_copy(
                    wbuf, out_hbm.at[wid, pl.ds((w - 1) * _WUS, _WUS)])

    wait_emb(buf_r1, sem_r1)
    accumulate(_NWV - 1, (_NWV - 1) * _WCD, buf_r1, 0)
    wait_emb(buf_s1, sem_s1)
    accumulate(_NWV - 1, _CPW + (_NWV - 1) * _WCD, buf_s1, _EMB_DIM)
    pltpu.sync_copy(wbuf, out_hbm.at[wid, pl.ds((_NWV - 1) * _WUS, _WUS)])


def kernel(user_ids, user2rec_code, user2src_code, code_embedding):
    mesh = plsc.VectorSubcoreMesh(core_axis_name="c", subcore_axis_name="s")
    out = pl.kernel(
        _body,
        out_type=jax.ShapeDtypeStruct((_NW, _UPW, 2 * _EMB_DIM), jnp.float32),
        mesh=mesh,
        compiler_params=pltpu.CompilerParams(
            needs_layout_passes=False, disable_bounds_checks=True),
        scratch_types=[
            pltpu.VMEM((_UPW + _LANES,), jnp.int32),
            pltpu.VMEM((_UPW + _LANES,), jnp.int32),
            pltpu.VMEM((_WUS, _CODE_LEN, 128), jnp.int32),
            pltpu.VMEM((_WUS, _CODE_LEN, 128), jnp.int32),
            pltpu.VMEM((_WUS, _CODE_LEN, 128), jnp.int32),
            pltpu.VMEM((_WUS, _CODE_LEN, 128), jnp.int32),
            pltpu.VMEM((2 * _CPW + _LANES,), jnp.int32),
            pltpu.VMEM((_WCD, _EMB_DIM), jnp.float32),
            pltpu.VMEM((_WCD, _EMB_DIM), jnp.float32),
            pltpu.VMEM((_WCD, _EMB_DIM), jnp.float32),
            pltpu.VMEM((_WCD, _EMB_DIM), jnp.float32),
            pltpu.VMEM((_WUS, 2 * _EMB_DIM), jnp.float32),
            pltpu.VMEM((_EMB_DIM,), jnp.float32),
            pltpu.SemaphoreType.DMA,
            pltpu.SemaphoreType.DMA,
            pltpu.SemaphoreType.DMA,
            pltpu.SemaphoreType.DMA,
            pltpu.SemaphoreType.DMA,
            pltpu.SemaphoreType.DMA,
        ],
    )(user_ids, user2rec_code.T, user2src_code.T, code_embedding)
    return out.reshape(_BATCH, 2 * _EMB_DIM)
